# baseline (speedup 1.0000x reference)
import functools

import jax
import jax.numpy as jnp
from jax import lax
from jax.experimental import pallas as pl
from jax.experimental.pallas import tpu as pltpu
from jax.experimental.pallas import tpu_sc as plsc

N_NODES = 10000
N_PAD = 10240
N_EDGES = 320000
DIM = 128
NC = 2
NS = 16
NW = NC * NS
LANES = 16
WIN = 64
WPW = 160
E_PAD = NW * WPW * WIN
EPW = WPW * WIN
NBUF = 4
ROWS_PER_TILE = N_PAD // NS
MM_BLK = 1000
ADJ_SCALE = 32768.0


def _matmul_body(x_ref, w_ref, o_ref):
    o_ref[...] = jnp.dot(x_ref[...], w_ref[...],
                         preferred_element_type=jnp.float32)


def _sum_body(a_ref, b_ref, o_ref):
    o_ref[...] = a_ref[...] + b_ref[...]


_mesh = plsc.VectorSubcoreMesh(core_axis_name="c", subcore_axis_name="s")


@functools.partial(
    pl.kernel,
    out_type=jax.ShapeDtypeStruct((NC, N_PAD, DIM), jnp.float32),
    mesh=_mesh,
    compiler_params=pltpu.CompilerParams(needs_layout_passes=False),
    scratch_types=[
        pltpu.VMEM((EPW,), jnp.int32),
        pltpu.VMEM((EPW // 2,), jnp.int32),
        pltpu.VMEM((WIN, DIM), jnp.float32),
        pltpu.VMEM((WIN, DIM), jnp.float32),
        pltpu.VMEM((WIN, DIM), jnp.float32),
        pltpu.VMEM((WIN, DIM), jnp.float32),
        pltpu.VMEM((WIN,), jnp.int32),
        pltpu.VMEM((WIN,), jnp.int32),
        pltpu.VMEM((WIN,), jnp.int32),
        pltpu.VMEM((WIN,), jnp.int32),
        pltpu.VMEM((WIN,), jnp.int32),
        pltpu.VMEM((WIN,), jnp.int32),
        pltpu.VMEM((WIN,), jnp.int32),
        pltpu.VMEM((WIN,), jnp.int32),
        pltpu.VMEM_SHARED((N_PAD, DIM), jnp.float32),
        pltpu.SemaphoreType.DMA,
        pltpu.SemaphoreType.DMA,
        pltpu.SemaphoreType.DMA,
        pltpu.SemaphoreType.DMA,
        pltpu.SemaphoreType.DMA,
        pltpu.SemaphoreType.DMA,
        pltpu.SemaphoreType.DMA,
        pltpu.SemaphoreType.DMA,
    ],
)
def _spmm_sc(x_hbm, packed_hbm, adj_hbm, zeros_hbm, part_hbm,
             packed_v, adj_v, r0, r1, r2, r3,
             si0, si1, si2, si3, di0, di1, di2, di3, acc_sh,
             g0, g1, g2, g3, s0, s1, s2, s3):
    rows = (r0, r1, r2, r3)
    sidx = (si0, si1, si2, si3)
    didx = (di0, di1, di2, di3)
    gsem = (g0, g1, g2, g3)
    ssem = (s0, s1, s2, s3)

    cid = lax.axis_index("c")
    sid = lax.axis_index("s")
    wid = sid * NC + cid
    row0 = sid * ROWS_PER_TILE

    pltpu.sync_copy(zeros_hbm.at[pl.ds(row0, ROWS_PER_TILE)],
                    acc_sh.at[pl.ds(row0, ROWS_PER_TILE)])
    pltpu.sync_copy(packed_hbm.at[wid], packed_v)
    pltpu.sync_copy(adj_hbm.at[wid], adj_v)
    plsc.subcore_barrier()

    def unpack_sidx(w, b):
        for g in range(WIN // LANES):
            v = packed_v[pl.ds(w * WIN + g * LANES, LANES)]
            sidx[b][pl.ds(g * LANES, LANES)] = lax.bitwise_and(v, 0xFFFF)

    def unpack_didx(w, b):
        for g in range(WIN // LANES):
            v = packed_v[pl.ds(w * WIN + g * LANES, LANES)]
            didx[b][pl.ds(g * LANES, LANES)] = lax.shift_right_logical(v, 16)

    def start_gather(w, b):
        pltpu.async_copy(x_hbm.at[sidx[b]], rows[b], gsem[b])

    def wait_gather(w, b):
        pltpu.make_async_copy(x_hbm.at[sidx[b]], rows[b], gsem[b]).wait()

    def start_scatter(w, b):
        pltpu.async_copy(rows[b], acc_sh.at[didx[b]], ssem[b], add=True)

    def wait_scatter(w, b):
        pltpu.make_async_copy(rows[b], acc_sh.at[didx[b]], ssem[b]).wait()

    def scale(w, rbuf):
        @plsc.parallel_loop(0, WIN // 8, unroll=4)
        def group(g):
            pair_base = jnp.full((LANES,), (w * WIN) // 2 + g * 4, jnp.int32)
            for l in range(8):
                pair = plsc.load_gather(adj_v, [pair_base + (l >> 1)])
                if l & 1:
                    a_i = lax.shift_right_logical(pair, 16)
                else:
                    a_i = lax.bitwise_and(pair, 0xFFFF)
                spl = a_i.astype(jnp.float32)
                row = g * 8 + l
                for j in range(DIM // LANES):
                    sl = pl.ds(j * LANES, LANES)
                    rbuf[row, sl] = rbuf[row, sl] * spl

    def body(w, b, b2, skip_ssem_wait=False, pf_gather=True):
        wait_gather(w, b)
        scale(w, rows[b])
        start_scatter(w, b)
        if pf_gather:
            if not skip_ssem_wait:
                wait_scatter(w - 2, b2)
            unpack_sidx(w + 2, b2)
            unpack_didx(w + 2, b2)
            start_gather(w + 2, b2)

    unpack_sidx(0, 0)
    unpack_didx(0, 0)
    unpack_sidx(1, 1)
    unpack_didx(1, 1)
    start_gather(0, 0)
    start_gather(1, 1)
    body(0, 0, 2, skip_ssem_wait=True)
    body(1, 1, 3, skip_ssem_wait=True)

    @pl.loop(2, WPW - 6, step=NBUF)
    def main(wb):
        for k in range(NBUF):
            b = (2 + k) % NBUF
            body(wb + k, b, k % NBUF)

    for k in range(6):
        w = WPW - 6 + k
        b = (2 + k) % NBUF
        body(w, b, k % NBUF, pf_gather=(k < 4))
    for k in range(NBUF):
        w = WPW - NBUF + k
        wait_scatter(w, w % NBUF)

    plsc.subcore_barrier()
    pltpu.sync_copy(acc_sh.at[pl.ds(row0, ROWS_PER_TILE)],
                    part_hbm.at[cid].at[pl.ds(row0, ROWS_PER_TILE)])


def kernel(inputs, edge_index, adj_values, W):
    src = edge_index[0].astype(jnp.int32)
    dst = edge_index[1].astype(jnp.int32)
    adj = adj_values.astype(jnp.float32)

    pad = E_PAD - N_EDGES
    pad_idx = (jnp.arange(pad, dtype=jnp.int32) * 97) % N_NODES
    src_p = jnp.concatenate([src, pad_idx])
    dst_p = jnp.concatenate([dst, pad_idx])
    packed = jnp.bitwise_or(jnp.left_shift(dst_p, 16), src_p)
    packed = packed.reshape(NW, EPW)
    adj_p = jnp.concatenate([adj, jnp.zeros((pad,), jnp.float32)])
    adjq = jnp.round(adj_p * ADJ_SCALE).astype(jnp.int32).reshape(NW, EPW)
    adj_pairs = jnp.bitwise_or(adjq[:, 0::2],
                               jnp.left_shift(adjq[:, 1::2], 16))
    zeros = jnp.zeros((N_PAD, DIM), jnp.float32)

    w2 = W * (1.0 / ADJ_SCALE)

    x = pl.pallas_call(
        _matmul_body,
        grid=(N_NODES // MM_BLK,),
        in_specs=[pl.BlockSpec((MM_BLK, DIM), lambda i: (i, 0)),
                  pl.BlockSpec((DIM, DIM), lambda i: (0, 0))],
        out_specs=pl.BlockSpec((MM_BLK, DIM), lambda i: (i, 0)),
        out_shape=jax.ShapeDtypeStruct((N_NODES, DIM), jnp.float32),
    )(inputs, w2)

    part = _spmm_sc(x, packed, adj_pairs, zeros)

    out = pl.pallas_call(
        _sum_body,
        grid=(N_NODES // MM_BLK,),
        in_specs=[pl.BlockSpec((MM_BLK, DIM), lambda i: (i, 0)),
                  pl.BlockSpec((MM_BLK, DIM), lambda i: (i, 0))],
        out_specs=pl.BlockSpec((MM_BLK, DIM), lambda i: (i, 0)),
        out_shape=jax.ShapeDtypeStruct((N_NODES, DIM), jnp.float32),
    )(part[0], part[1])
    return out


# speedup vs baseline: 1.3197x; 1.3197x over previous
import functools

import jax
import jax.numpy as jnp
from jax import lax
from jax.experimental import pallas as pl
from jax.experimental.pallas import tpu as pltpu
from jax.experimental.pallas import tpu_sc as plsc

N_NODES = 10000
N_PAD = 10240
N_EDGES = 320000
DIM = 128
NC = 2
NS = 16
NW = NC * NS
LANES = 16
WIN = 64
WPW = 160
E_PAD = NW * WPW * WIN
EPW = WPW * WIN
NBUF = 3
ROWS_PER_TILE = N_PAD // NS
MM_BLK = 1000


def _matmul_body(x_ref, w_ref, o_ref):
    o_ref[...] = jnp.dot(x_ref[...], w_ref[...],
                         preferred_element_type=jnp.float32)


def _sum_body(a_ref, b_ref, o_ref):
    o_ref[...] = a_ref[...] + b_ref[...]


_mesh = plsc.VectorSubcoreMesh(core_axis_name="c", subcore_axis_name="s")


@functools.partial(
    pl.kernel,
    out_type=jax.ShapeDtypeStruct((NC, N_PAD, DIM), jnp.float32),
    mesh=_mesh,
    compiler_params=pltpu.CompilerParams(needs_layout_passes=False),
    scratch_types=[
        pltpu.VMEM((EPW,), jnp.int32),
        pltpu.VMEM((EPW,), jnp.float32),
        pltpu.VMEM((WIN, DIM), jnp.float32),
        pltpu.VMEM((WIN, DIM), jnp.float32),
        pltpu.VMEM((WIN, DIM), jnp.float32),
        pltpu.VMEM((WIN,), jnp.int32),
        pltpu.VMEM((WIN,), jnp.int32),
        pltpu.VMEM((WIN,), jnp.int32),
        pltpu.VMEM((WIN,), jnp.int32),
        pltpu.VMEM((WIN,), jnp.int32),
        pltpu.VMEM((WIN,), jnp.int32),
        pltpu.VMEM_SHARED((N_PAD, DIM), jnp.float32),
        pltpu.SemaphoreType.DMA,
        pltpu.SemaphoreType.DMA,
        pltpu.SemaphoreType.DMA,
        pltpu.SemaphoreType.DMA,
        pltpu.SemaphoreType.DMA,
        pltpu.SemaphoreType.DMA,
    ],
)
def _spmm_sc(x_hbm, packed_hbm, adj_hbm, zeros_hbm, part_hbm,
             packed_v, adj_v, r0, r1, r2,
             si0, si1, si2, di0, di1, di2, acc_sh,
             g0, g1, g2, s0, s1, s2):
    rows = (r0, r1, r2)
    sidx = (si0, si1, si2)
    didx = (di0, di1, di2)
    gsem = (g0, g1, g2)
    ssem = (s0, s1, s2)

    cid = lax.axis_index("c")
    sid = lax.axis_index("s")
    wid = sid * NC + cid
    row0 = sid * ROWS_PER_TILE

    pltpu.sync_copy(zeros_hbm.at[pl.ds(row0, ROWS_PER_TILE)],
                    acc_sh.at[pl.ds(row0, ROWS_PER_TILE)])
    pltpu.sync_copy(packed_hbm.at[wid], packed_v)
    pltpu.sync_copy(adj_hbm.at[wid], adj_v)
    plsc.subcore_barrier()

    def unpack(w, b):
        for g in range(WIN // LANES):
            v = packed_v[pl.ds(w * WIN + g * LANES, LANES)]
            sl = pl.ds(g * LANES, LANES)
            sidx[b][sl] = lax.bitwise_and(v, 0xFFFF)
            didx[b][sl] = lax.shift_right_logical(v, 16)

    def start_gather(w, b):
        pltpu.async_copy(x_hbm.at[sidx[b]], rows[b], gsem[b])

    def wait_gather(w, b):
        pltpu.make_async_copy(x_hbm.at[sidx[b]], rows[b], gsem[b]).wait()

    def start_scatter(w, b):
        pltpu.async_copy(rows[b], acc_sh.at[didx[b]], ssem[b], add=True)

    def wait_scatter(w, b):
        pltpu.make_async_copy(rows[b], acc_sh.at[didx[b]], ssem[b]).wait()

    def scale(w, rbuf):
        @plsc.parallel_loop(0, WIN // 8, unroll=4)
        def group(g):
            base_vec = jnp.full((LANES,), w * WIN + g * 8, jnp.int32)
            for l in range(8):
                spl = plsc.load_gather(adj_v, [base_vec + l])
                row = g * 8 + l
                for j in range(DIM // LANES):
                    sl = pl.ds(j * LANES, LANES)
                    rbuf[row, sl] = rbuf[row, sl] * spl

    def body(w, b, b2, first=False, prefetch=True):
        wait_gather(w, b)
        scale(w, rows[b])
        start_scatter(w, b)
        if prefetch:
            if not first:
                wait_scatter(w - 1, b2)
            unpack(w + 2, b2)
            start_gather(w + 2, b2)

    unpack(0, 0)
    unpack(1, 1)
    start_gather(0, 0)
    start_gather(1, 1)
    body(0, 0, 2, first=True)
    body(1, 1, 0)

    @pl.loop(2, WPW - 8, step=NBUF)
    def main(wb):
        for k in range(NBUF):
            w = wb + k
            b = (2 + k) % NBUF
            body(w, b, (b + 2) % NBUF)

    for k in range(8):
        w = WPW - 8 + k
        b = (2 + k) % NBUF
        body(w, b, (b + 2) % NBUF, prefetch=(k < 6))
    for k in range(NBUF):
        w = WPW - NBUF + k
        wait_scatter(w, w % NBUF)

    plsc.subcore_barrier()
    pltpu.sync_copy(acc_sh.at[pl.ds(row0, ROWS_PER_TILE)],
                    part_hbm.at[cid].at[pl.ds(row0, ROWS_PER_TILE)])


def kernel(inputs, edge_index, adj_values, W):
    src = edge_index[0].astype(jnp.int32)
    dst = edge_index[1].astype(jnp.int32)
    adj = adj_values.astype(jnp.float32)

    pad = E_PAD - N_EDGES
    pad_idx = (jnp.arange(pad, dtype=jnp.int32) * 97) % N_NODES
    src_p = jnp.concatenate([src, pad_idx])
    dst_p = jnp.concatenate([dst, pad_idx])
    packed = jnp.bitwise_or(jnp.left_shift(dst_p, 16), src_p)
    packed = packed.reshape(NW, EPW)
    adj_p = jnp.concatenate([adj, jnp.zeros((pad,), jnp.float32)])
    adj_p = adj_p.reshape(NW, EPW)
    zeros = jnp.zeros((N_PAD, DIM), jnp.float32)

    x = pl.pallas_call(
        _matmul_body,
        grid=(N_NODES // MM_BLK,),
        in_specs=[pl.BlockSpec((MM_BLK, DIM), lambda i: (i, 0)),
                  pl.BlockSpec((DIM, DIM), lambda i: (0, 0))],
        out_specs=pl.BlockSpec((MM_BLK, DIM), lambda i: (i, 0)),
        out_shape=jax.ShapeDtypeStruct((N_NODES, DIM), jnp.float32),
    )(inputs, W)

    part = _spmm_sc(x, packed, adj_p, zeros)

    out = pl.pallas_call(
        _sum_body,
        grid=(N_NODES // MM_BLK,),
        in_specs=[pl.BlockSpec((MM_BLK, DIM), lambda i: (i, 0)),
                  pl.BlockSpec((MM_BLK, DIM), lambda i: (i, 0))],
        out_specs=pl.BlockSpec((MM_BLK, DIM), lambda i: (i, 0)),
        out_shape=jax.ShapeDtypeStruct((N_NODES, DIM), jnp.float32),
    )(part[0], part[1])
    return out


# speedup vs baseline: 1.3315x; 1.0089x over previous
import functools

import jax
import jax.numpy as jnp
from jax import lax
from jax.experimental import pallas as pl
from jax.experimental.pallas import tpu as pltpu
from jax.experimental.pallas import tpu_sc as plsc

N_NODES = 10000
N_PAD = 10240
N_EDGES = 320000
DIM = 128
NC = 2
NS = 16
NW = NC * NS
LANES = 16
WIN = 64
WPW = 160
E_PAD = NW * WPW * WIN
EPW = WPW * WIN
NBUF = 3
ROWS_PER_TILE = N_PAD // NS
MM_BLK = 1000


def _matmul_body(x_ref, w_ref, o_ref):
    o_ref[...] = jnp.dot(x_ref[...], w_ref[...],
                         preferred_element_type=jnp.float32)


def _sum_body(a_ref, b_ref, o_ref):
    o_ref[...] = a_ref[...] + b_ref[...]


_mesh = plsc.VectorSubcoreMesh(core_axis_name="c", subcore_axis_name="s")


@functools.partial(
    pl.kernel,
    out_type=jax.ShapeDtypeStruct((NC, N_PAD, DIM), jnp.float32),
    mesh=_mesh,
    compiler_params=pltpu.CompilerParams(needs_layout_passes=False),
    scratch_types=[
        pltpu.VMEM((EPW,), jnp.int32),
        pltpu.VMEM((EPW,), jnp.float32),
        pltpu.VMEM((WIN, DIM), jnp.float32),
        pltpu.VMEM((WIN, DIM), jnp.float32),
        pltpu.VMEM((WIN, DIM), jnp.float32),
        pltpu.VMEM((WIN,), jnp.int32),
        pltpu.VMEM((WIN,), jnp.int32),
        pltpu.VMEM((WIN,), jnp.int32),
        pltpu.VMEM((WIN,), jnp.int32),
        pltpu.VMEM((WIN,), jnp.int32),
        pltpu.VMEM((WIN,), jnp.int32),
        pltpu.VMEM_SHARED((N_PAD, DIM), jnp.float32),
        pltpu.SemaphoreType.DMA,
        pltpu.SemaphoreType.DMA,
        pltpu.SemaphoreType.DMA,
        pltpu.SemaphoreType.DMA,
        pltpu.SemaphoreType.DMA,
        pltpu.SemaphoreType.DMA,
    ],
)
def _spmm_sc(x_hbm, packed_hbm, adj_hbm, zeros_hbm, part_hbm,
             packed_v, adj_v, r0, r1, r2,
             si0, si1, si2, di0, di1, di2, acc_sh,
             g0, g1, g2, s0, s1, s2):
    rows = (r0, r1, r2)
    sidx = (si0, si1, si2)
    didx = (di0, di1, di2)
    gsem = (g0, g1, g2)
    ssem = (s0, s1, s2)

    cid = lax.axis_index("c")
    sid = lax.axis_index("s")
    wid = sid * NC + cid
    row0 = sid * ROWS_PER_TILE

    z_src = zeros_hbm.at[pl.ds(row0, ROWS_PER_TILE)]
    z_dst = acc_sh.at[pl.ds(row0, ROWS_PER_TILE)]
    pltpu.async_copy(z_src, z_dst, g0)
    pltpu.async_copy(packed_hbm.at[wid], packed_v, g1)
    pltpu.async_copy(adj_hbm.at[wid], adj_v, g2)
    pltpu.make_async_copy(z_src, z_dst, g0).wait()
    pltpu.make_async_copy(packed_hbm.at[wid], packed_v, g1).wait()
    pltpu.make_async_copy(adj_hbm.at[wid], adj_v, g2).wait()
    plsc.subcore_barrier()

    def unpack(w, b):
        for g in range(WIN // LANES):
            v = packed_v[pl.ds(w * WIN + g * LANES, LANES)]
            sl = pl.ds(g * LANES, LANES)
            sidx[b][sl] = lax.bitwise_and(v, 0xFFFF)
            didx[b][sl] = lax.shift_right_logical(v, 16)

    def start_gather(w, b):
        pltpu.async_copy(x_hbm.at[sidx[b]], rows[b], gsem[b])

    def wait_gather(w, b):
        pltpu.make_async_copy(x_hbm.at[sidx[b]], rows[b], gsem[b]).wait()

    def start_scatter(w, b):
        pltpu.async_copy(rows[b], acc_sh.at[didx[b]], ssem[b], add=True)

    def wait_scatter(w, b):
        pltpu.make_async_copy(rows[b], acc_sh.at[didx[b]], ssem[b]).wait()

    def scale(w, rbuf):
        @plsc.parallel_loop(0, WIN // 8, unroll=4)
        def group(g):
            base_vec = jnp.full((LANES,), w * WIN + g * 8, jnp.int32)
            for l in range(8):
                spl = plsc.load_gather(adj_v, [base_vec + l])
                row = g * 8 + l
                for j in range(DIM // LANES):
                    sl = pl.ds(j * LANES, LANES)
                    rbuf[row, sl] = rbuf[row, sl] * spl

    def body(w, b, b2, first=False, prefetch=True):
        wait_gather(w, b)
        scale(w, rows[b])
        start_scatter(w, b)
        if prefetch:
            if not first:
                wait_scatter(w - 1, b2)
            unpack(w + 2, b2)
            start_gather(w + 2, b2)

    unpack(0, 0)
    unpack(1, 1)
    start_gather(0, 0)
    start_gather(1, 1)
    body(0, 0, 2, first=True)
    body(1, 1, 0)

    @pl.loop(2, WPW - 8, step=NBUF)
    def main(wb):
        for k in range(NBUF):
            w = wb + k
            b = (2 + k) % NBUF
            body(w, b, (b + 2) % NBUF)

    for k in range(8):
        w = WPW - 8 + k
        b = (2 + k) % NBUF
        body(w, b, (b + 2) % NBUF, prefetch=(k < 6))
    for k in range(NBUF):
        w = WPW - NBUF + k
        wait_scatter(w, w % NBUF)

    plsc.subcore_barrier()
    pltpu.sync_copy(acc_sh.at[pl.ds(row0, ROWS_PER_TILE)],
                    part_hbm.at[cid].at[pl.ds(row0, ROWS_PER_TILE)])


def kernel(inputs, edge_index, adj_values, W):
    src = edge_index[0].astype(jnp.int32)
    dst = edge_index[1].astype(jnp.int32)
    adj = adj_values.astype(jnp.float32)

    pad = E_PAD - N_EDGES
    pad_idx = (jnp.arange(pad, dtype=jnp.int32) * 97) % N_NODES
    src_p = jnp.concatenate([src, pad_idx])
    dst_p = jnp.concatenate([dst, pad_idx])
    packed = jnp.bitwise_or(jnp.left_shift(dst_p, 16), src_p)
    packed = packed.reshape(NW, EPW)
    adj_p = jnp.concatenate([adj, jnp.zeros((pad,), jnp.float32)])
    adj_p = adj_p.reshape(NW, EPW)
    zeros = jnp.zeros((N_PAD, DIM), jnp.float32)

    x = pl.pallas_call(
        _matmul_body,
        grid=(N_NODES // MM_BLK,),
        in_specs=[pl.BlockSpec((MM_BLK, DIM), lambda i: (i, 0)),
                  pl.BlockSpec((DIM, DIM), lambda i: (0, 0))],
        out_specs=pl.BlockSpec((MM_BLK, DIM), lambda i: (i, 0)),
        out_shape=jax.ShapeDtypeStruct((N_NODES, DIM), jnp.float32),
    )(inputs, W)

    part = _spmm_sc(x, packed, adj_p, zeros)

    out = pl.pallas_call(
        _sum_body,
        grid=(N_NODES // MM_BLK,),
        in_specs=[pl.BlockSpec((MM_BLK, DIM), lambda i: (i, 0)),
                  pl.BlockSpec((MM_BLK, DIM), lambda i: (i, 0))],
        out_specs=pl.BlockSpec((MM_BLK, DIM), lambda i: (i, 0)),
        out_shape=jax.ShapeDtypeStruct((N_NODES, DIM), jnp.float32),
    )(part[0], part[1])
    return out
